# baseline (speedup 1.0000x reference)
import jax
import jax.numpy as jnp
from jax import lax
from jax.experimental import pallas as pl
from jax.experimental.pallas import tpu as pltpu
from jax.experimental.pallas import tpu_sc as plsc

HIDDEN = 64
KV = HIDDEN // 16
TYPE_VOCAB = 5
B = 4096
L = 50
EPS = 1e-12
W = 128
LP = 56

NC = 2
NS = 16
LANES = 16
NW = NC * NS
BROWS_PER_W = B // NW
CB = 8
NFULL = CB * 3
XROWS = NFULL * LANES
NCHUNK = BROWS_PER_W // CB
POS_STAGE = 56


def _take(vec, idx):
    return lax.gather(
        vec, idx[:, None],
        dimension_numbers=lax.GatherDimensionNumbers(
            offset_dims=(), collapsed_slice_dims=(0,), start_index_map=(0,)),
        slice_sizes=(1,),
        mode=lax.GatherScatterMode.PROMISE_IN_BOUNDS)


def _rsqrt(v):
    i = plsc.bitcast(v, jnp.int32)
    i = jnp.int32(0x5F3759DF) - (i >> 1)
    y = plsc.bitcast(i, jnp.float32)
    for _ in range(3):
        y = y * (1.5 - 0.5 * v * y * y)
    return y


def _body(pk_hbm, word_hbm, pos_hbm, typ_hbm, gam_hbm, bet_hbm,
          out_hbm, pk_v, x_v, xt_v, y_v, pos_v, typ_v, gam_v, bet_v,
          sem, sem_o):
    wid = lax.axis_index("s") * NC + lax.axis_index("c")
    brow0 = wid * BROWS_PER_W

    pltpu.sync_copy(pos_hbm.at[pl.ds(0, POS_STAGE)], pos_v)
    pltpu.sync_copy(typ_hbm, typ_v)
    pltpu.sync_copy(gam_hbm, gam_v)
    pltpu.sync_copy(bet_hbm, bet_v)

    gv = [gam_v[pl.ds(k * LANES, LANES)] for k in range(KV)]
    bv = [bet_v[pl.ds(k * LANES, LANES)] for k in range(KV)]
    idx15 = jnp.full((LANES,), LANES - 1, jnp.int32)
    iota = lax.iota(jnp.int32, LANES)

    def _token_group(xref, tvec, rows, lps, ysts):
        for j in range(LANES):
            row = rows[j]
            xs = [xref[row, pl.ds(k * LANES, LANES)] for k in range(KV)]
            ps = [pos_v[lps[j], pl.ds(k * LANES, LANES)] for k in range(KV)]
            tsp = _take(tvec, jnp.full((LANES,), j, jnp.int32))
            vs = []
            for k in range(KV):
                tk = plsc.load_gather(typ_v, [tsp, k * LANES + iota])
                vs.append(xs[k] + ps[k] + tk)
            s = (vs[0] + vs[1]) + (vs[2] + vs[3])
            q = (vs[0] * vs[0] + vs[1] * vs[1]) + (
                vs[2] * vs[2] + vs[3] * vs[3])
            tot = _take(plsc.cumsum(s), idx15)
            totq = _take(plsc.cumsum(q), idx15)
            mean = tot * (1.0 / HIDDEN)
            var = totq * (1.0 / HIDDEN) - mean * mean
            r = _rsqrt(var + EPS)
            yr, yl = ysts[j]
            for k in range(KV):
                y_v[yr, yl, pl.ds(k * LANES, LANES)] = (
                    (vs[k] - mean) * r * gv[k] + bv[k])

    pltpu.async_copy(y_v, out_hbm.at[pl.ds(brow0, CB)], sem_o)

    def _chunk(c, carry):
        b0 = brow0 + c * CB
        pltpu.sync_copy(pk_hbm.at[pl.ds(b0, CB)], pk_v)
        cps = []
        for rr in range(CB):
            for oi in range(3):
                iv = lax.shift_right_logical(
                    pk_v[rr, pl.ds(oi * LANES, LANES)], 3)
                cps.append(pltpu.async_copy(
                    word_hbm.at[iv],
                    x_v.at[pl.ds((rr * 3 + oi) * LANES, LANES)], sem))
        ivt = lax.shift_right_logical(
            plsc.load_gather(
                pk_v, [iota & (CB - 1),
                       3 * LANES + lax.shift_right_logical(iota, 3)]), 3)
        cps.append(pltpu.async_copy(word_hbm.at[ivt], xt_v, sem))
        pltpu.make_async_copy(y_v, out_hbm.at[pl.ds(b0, CB)], sem_o).wait()

        def _full(w):
            r = lax.shift_right_logical(w * 21846, 16)
            o = (w - r * 3) * LANES
            tvec = plsc.load_gather(
                pk_v, [jnp.full((LANES,), r, jnp.int32), o + iota]) & 7
            rb = w * LANES
            _token_group(
                x_v, tvec,
                rows=[rb + j for j in range(LANES)],
                lps=[o + j for j in range(LANES)],
                ysts=[(r, o + j) for j in range(LANES)])

        half = NFULL // 2
        for cp in cps[:half]:
            cp.wait()
        plsc.parallel_loop(0, half, 1)(_full)
        for cp in cps[half:]:
            cp.wait()
        plsc.parallel_loop(half, NFULL, 1)(_full)

        tvec = plsc.load_gather(
            pk_v, [iota & (CB - 1),
                   3 * LANES + lax.shift_right_logical(iota, 3)]) & 7
        _token_group(
            xt_v, tvec,
            rows=list(range(LANES)),
            lps=[3 * LANES + j // CB for j in range(LANES)],
            ysts=[(j % CB, 3 * LANES + j // CB) for j in range(LANES)])

        pltpu.async_copy(y_v, out_hbm.at[pl.ds(b0, CB)], sem_o)
        return carry

    lax.fori_loop(0, NCHUNK, _chunk, 0)
    pltpu.make_async_copy(
        y_v, out_hbm.at[pl.ds(brow0 + (NCHUNK - 1) * CB, CB)], sem_o).wait()


def kernel(input_ids, input_mask, word_table, pos_table, type_table, gamma, beta):
    packed = jnp.pad((input_ids << 3) | input_mask, ((0, 0), (0, W - L)))
    w128 = jnp.pad(word_table, ((0, 0), (0, W - HIDDEN)))
    pos128 = jnp.pad(pos_table, ((0, 0), (0, W - HIDDEN)))
    typ128 = jnp.pad(type_table, ((0, 3), (0, W - HIDDEN)))
    gam128 = jnp.pad(gamma, (0, W - HIDDEN))
    bet128 = jnp.pad(beta, (0, W - HIDDEN))

    mesh = plsc.VectorSubcoreMesh(
        core_axis_name="c", subcore_axis_name="s",
        num_cores=NC, num_subcores=NS)
    f = pl.kernel(
        _body,
        out_type=jax.ShapeDtypeStruct((B, LP, W), jnp.float32),
        mesh=mesh,
        compiler_params=pltpu.CompilerParams(
            needs_layout_passes=False, use_tc_tiling_on_sc=True),
        scratch_types=[
            pltpu.VMEM((CB, W), jnp.int32),
            pltpu.VMEM((XROWS, W), jnp.float32),
            pltpu.VMEM((LANES, W), jnp.float32),
            pltpu.VMEM((CB, LP, W), jnp.float32),
            pltpu.VMEM((POS_STAGE, W), jnp.float32),
            pltpu.VMEM((8, W), jnp.float32),
            pltpu.VMEM((W,), jnp.float32),
            pltpu.VMEM((W,), jnp.float32),
            pltpu.SemaphoreType.DMA,
            pltpu.SemaphoreType.DMA,
        ],
    )
    out = f(packed, w128, pos128, typ128, gam128, bet128)
    return out[:, :L, :HIDDEN]


# speedup vs baseline: 1.0143x; 1.0143x over previous
import jax
import jax.numpy as jnp
from jax import lax
from jax.experimental import pallas as pl
from jax.experimental.pallas import tpu as pltpu
from jax.experimental.pallas import tpu_sc as plsc

HIDDEN = 64
KV = HIDDEN // 16
TYPE_VOCAB = 5
B = 4096
L = 50
EPS = 1e-12
W = 128
LP = 56

NC = 2
NS = 16
LANES = 16
NW = NC * NS
BROWS_PER_W = B // NW
CB = 8
NFULL = CB * 3
XROWS = NFULL * LANES
NCHUNK = BROWS_PER_W // CB
POS_STAGE = 56


def _take(vec, idx):
    return lax.gather(
        vec, idx[:, None],
        dimension_numbers=lax.GatherDimensionNumbers(
            offset_dims=(), collapsed_slice_dims=(0,), start_index_map=(0,)),
        slice_sizes=(1,),
        mode=lax.GatherScatterMode.PROMISE_IN_BOUNDS)


def _rsqrt(v):
    i = plsc.bitcast(v, jnp.int32)
    i = jnp.int32(0x5F3759DF) - (i >> 1)
    y = plsc.bitcast(i, jnp.float32)
    for _ in range(3):
        y = y * (1.5 - 0.5 * v * y * y)
    return y


def _body(pk_hbm, word_hbm, pos_hbm, typ_hbm, gam_hbm, bet_hbm,
          out_hbm, pk_v, x_v, xt_v, y_v, pos_v, typ_v, gam_v, bet_v,
          sem, sem_o):
    wid = lax.axis_index("s") * NC + lax.axis_index("c")
    brow0 = wid * BROWS_PER_W

    pltpu.sync_copy(pos_hbm.at[pl.ds(0, POS_STAGE)], pos_v)
    pltpu.sync_copy(typ_hbm, typ_v)
    pltpu.sync_copy(gam_hbm, gam_v)
    pltpu.sync_copy(bet_hbm, bet_v)

    tv = [[typ_v[t, pl.ds(k * LANES, LANES)] for k in range(KV)]
          for t in range(TYPE_VOCAB)]
    gv = [gam_v[pl.ds(k * LANES, LANES)] for k in range(KV)]
    bv = [bet_v[pl.ds(k * LANES, LANES)] for k in range(KV)]
    idx15 = jnp.full((LANES,), LANES - 1, jnp.int32)
    iota = lax.iota(jnp.int32, LANES)

    def _token_group(xref, tvec, rows, lps, ysts):
        for j in range(LANES):
            row = rows[j]
            xs = [xref[row, pl.ds(k * LANES, LANES)] for k in range(KV)]
            ps = [pos_v[lps[j], pl.ds(k * LANES, LANES)] for k in range(KV)]
            tsp = _take(tvec, jnp.full((LANES,), j, jnp.int32))
            m = [tsp == t for t in range(TYPE_VOCAB - 1)]
            vs = []
            for k in range(KV):
                tk = tv[TYPE_VOCAB - 1][k]
                for t in range(TYPE_VOCAB - 2, -1, -1):
                    tk = jnp.where(m[t], tv[t][k], tk)
                vs.append(xs[k] + ps[k] + tk)
            s = (vs[0] + vs[1]) + (vs[2] + vs[3])
            q = (vs[0] * vs[0] + vs[1] * vs[1]) + (
                vs[2] * vs[2] + vs[3] * vs[3])
            tot = _take(plsc.cumsum(s), idx15)
            totq = _take(plsc.cumsum(q), idx15)
            mean = tot * (1.0 / HIDDEN)
            var = totq * (1.0 / HIDDEN) - mean * mean
            r = _rsqrt(var + EPS)
            yr, yl = ysts[j]
            for k in range(KV):
                y_v[yr, yl, pl.ds(k * LANES, LANES)] = (
                    (vs[k] - mean) * r * gv[k] + bv[k])

    pltpu.async_copy(y_v, out_hbm.at[pl.ds(brow0, CB)], sem_o)

    def _chunk(c, carry):
        b0 = brow0 + c * CB
        pltpu.sync_copy(pk_hbm.at[pl.ds(b0, CB)], pk_v)
        cps = []
        for rr in range(CB):
            for oi in range(3):
                iv = lax.shift_right_logical(
                    pk_v[rr, pl.ds(oi * LANES, LANES)], 3)
                cps.append(pltpu.async_copy(
                    word_hbm.at[iv],
                    x_v.at[pl.ds((rr * 3 + oi) * LANES, LANES)], sem))
        ivt = lax.shift_right_logical(
            plsc.load_gather(
                pk_v, [iota & (CB - 1),
                       3 * LANES + lax.shift_right_logical(iota, 3)]), 3)
        cps.append(pltpu.async_copy(word_hbm.at[ivt], xt_v, sem))
        pltpu.make_async_copy(y_v, out_hbm.at[pl.ds(b0, CB)], sem_o).wait()

        def _full(w):
            r = lax.shift_right_logical(w * 21846, 16)
            o = (w - r * 3) * LANES
            tvec = plsc.load_gather(
                pk_v, [jnp.full((LANES,), r, jnp.int32), o + iota]) & 7
            rb = w * LANES
            _token_group(
                x_v, tvec,
                rows=[rb + j for j in range(LANES)],
                lps=[o + j for j in range(LANES)],
                ysts=[(r, o + j) for j in range(LANES)])

        half = NFULL // 2
        for cp in cps[:half]:
            cp.wait()
        plsc.parallel_loop(0, half, 1)(_full)
        for cp in cps[half:]:
            cp.wait()
        plsc.parallel_loop(half, NFULL, 1)(_full)

        tvec = plsc.load_gather(
            pk_v, [iota & (CB - 1),
                   3 * LANES + lax.shift_right_logical(iota, 3)]) & 7
        _token_group(
            xt_v, tvec,
            rows=list(range(LANES)),
            lps=[3 * LANES + j // CB for j in range(LANES)],
            ysts=[(j % CB, 3 * LANES + j // CB) for j in range(LANES)])

        pltpu.async_copy(y_v, out_hbm.at[pl.ds(b0, CB)], sem_o)
        return carry

    lax.fori_loop(0, NCHUNK, _chunk, 0)
    pltpu.make_async_copy(
        y_v, out_hbm.at[pl.ds(brow0 + (NCHUNK - 1) * CB, CB)], sem_o).wait()


def kernel(input_ids, input_mask, word_table, pos_table, type_table, gamma, beta):
    packed = jnp.pad((input_ids << 3) | input_mask, ((0, 0), (0, W - L)))
    w128 = jnp.pad(word_table, ((0, 0), (0, W - HIDDEN)))
    pos128 = jnp.pad(pos_table, ((0, 0), (0, W - HIDDEN)))
    typ128 = jnp.pad(type_table, ((0, 3), (0, W - HIDDEN)))
    gam128 = jnp.pad(gamma, (0, W - HIDDEN))
    bet128 = jnp.pad(beta, (0, W - HIDDEN))

    mesh = plsc.VectorSubcoreMesh(
        core_axis_name="c", subcore_axis_name="s",
        num_cores=NC, num_subcores=NS)
    f = pl.kernel(
        _body,
        out_type=jax.ShapeDtypeStruct((B, LP, W), jnp.float32),
        mesh=mesh,
        compiler_params=pltpu.CompilerParams(
            needs_layout_passes=False, use_tc_tiling_on_sc=True),
        scratch_types=[
            pltpu.VMEM((CB, W), jnp.int32),
            pltpu.VMEM((XROWS, W), jnp.float32),
            pltpu.VMEM((LANES, W), jnp.float32),
            pltpu.VMEM((CB, LP, W), jnp.float32),
            pltpu.VMEM((POS_STAGE, W), jnp.float32),
            pltpu.VMEM((8, W), jnp.float32),
            pltpu.VMEM((W,), jnp.float32),
            pltpu.VMEM((W,), jnp.float32),
            pltpu.SemaphoreType.DMA,
            pltpu.SemaphoreType.DMA,
        ],
    )
    out = f(packed, w128, pos128, typ128, gam128, bet128)
    return out[:, :L, :HIDDEN]
